# baseline (speedup 1.0000x reference)
import functools

import jax
import jax.numpy as jnp
import numpy as np
from jax.experimental import pallas as pl
from jax.experimental.pallas import tpu as pltpu

CARD = [119, 4, 12, 12, 10, 6, 6, 2, 2]
OFFS = [0, 119, 123, 135, 147, 157, 163, 169, 171]
TOT = 176
EMB = 48
HIDDEN = 256
BLOCK = 25000


def _project_body(e0, e1, e2, e3, e4, e5, e6, e7, e8, w_ref, b_ref, p_ref):
    embs = [e0, e1, e2, e3, e4, e5, e6, e7, e8]
    parts = []
    for i in range(9):
        wi = w_ref[EMB * i:EMB * (i + 1), :]
        pi = jnp.dot(embs[i][...], wi, preferred_element_type=jnp.float32,
                     precision=jax.lax.Precision.HIGHEST)
        parts.append(pi)
    parts[0] = parts[0] + b_ref[...]
    parts.append(jnp.zeros((TOT - sum(CARD), HIDDEN), jnp.float32))
    p_ref[...] = jnp.concatenate(parts, axis=0)


def _main_body(x_ref, s_ref, t_ref, p_ref, o_ref):
    xf = x_ref[0].astype(jnp.float32)
    xg = jax.lax.dot_general(xf, s_ref[...], (((0,), (0,)), ((), ())),
                             preferred_element_type=jnp.float32)
    m = jnp.where(xg == t_ref[...], 1.0, 0.0)
    h = jnp.dot(m, p_ref[...], preferred_element_type=jnp.float32)
    o_ref[...] = h * 0.5 * (1.0 + jax.lax.erf(h * np.float32(1.0 / np.sqrt(2.0))))


def _lane_consts():
    s = np.zeros((9, TOT), np.float32)
    t = np.full((1, TOT), -1.0, np.float32)
    for i, (off, c) in enumerate(zip(OFFS, CARD)):
        s[i, off:off + c] = 1.0
        t[0, off:off + c] = np.arange(c, dtype=np.float32)
    return jnp.asarray(s), jnp.asarray(t)


@functools.partial(jax.jit, static_argnames=())
def kernel(x, emb0, emb1, emb2, emb3, emb4, emb5, emb6, emb7, emb8, W, b):
    n = x.shape[0]
    p = pl.pallas_call(
        _project_body,
        out_shape=jax.ShapeDtypeStruct((TOT, HIDDEN), jnp.float32),
    )(emb0, emb1, emb2, emb3, emb4, emb5, emb6, emb7, emb8, W,
      b.reshape(1, HIDDEN))
    s, t = _lane_consts()
    grid = (n // BLOCK,)
    out = pl.pallas_call(
        _main_body,
        grid=grid,
        in_specs=[
            pl.BlockSpec((1, 9, BLOCK), lambda i: (i, 0, 0)),
            pl.BlockSpec((9, TOT), lambda i: (0, 0)),
            pl.BlockSpec((1, TOT), lambda i: (0, 0)),
            pl.BlockSpec((TOT, HIDDEN), lambda i: (0, 0)),
        ],
        out_specs=pl.BlockSpec((BLOCK, HIDDEN), lambda i: (i, 0)),
        out_shape=jax.ShapeDtypeStruct((n, HIDDEN), jnp.float32),
        compiler_params=pltpu.CompilerParams(
            dimension_semantics=("parallel",)),
    )(x.T.reshape(9, n // BLOCK, BLOCK).transpose(1, 0, 2), s, t, p)
    return out


# speedup vs baseline: 1.0288x; 1.0288x over previous
import functools

import jax
import jax.numpy as jnp
import numpy as np
from jax.experimental import pallas as pl
from jax.experimental.pallas import tpu as pltpu

CARD = [119, 4, 12, 12, 10, 6, 6, 2, 2]
OFFS = [0, 119, 123, 135, 147, 157, 163, 169, 171]
TOT = 176
EMB = 48
HIDDEN = 256
BLOCK = 25000


def _main_body(x_ref, s_ref, t_ref, e0, e1, e2, e3, e4, e5, e6, e7, e8,
               w_ref, b_ref, o_ref):
    embs = [e0, e1, e2, e3, e4, e5, e6, e7, e8]
    parts = []
    for i in range(9):
        wi = w_ref[EMB * i:EMB * (i + 1), :]
        pi = jnp.dot(embs[i][...], wi, preferred_element_type=jnp.float32,
                     precision=jax.lax.Precision.HIGHEST)
        parts.append(pi)
    parts[0] = parts[0] + b_ref[...]
    parts.append(jnp.zeros((TOT - sum(CARD), HIDDEN), jnp.float32))
    p = jnp.concatenate(parts, axis=0)

    xf = x_ref[0].astype(jnp.float32)
    xg = jax.lax.dot_general(xf, s_ref[...], (((0,), (0,)), ((), ())),
                             preferred_element_type=jnp.float32)
    m = jnp.where(xg == t_ref[...], 1.0, 0.0)
    h = jnp.dot(m, p, preferred_element_type=jnp.float32)
    o_ref[...] = h * 0.5 * (1.0 + jax.lax.erf(h * np.float32(1.0 / np.sqrt(2.0))))


def _lane_consts():
    s = np.zeros((9, TOT), np.float32)
    t = np.full((1, TOT), -1.0, np.float32)
    for i, (off, c) in enumerate(zip(OFFS, CARD)):
        s[i, off:off + c] = 1.0
        t[0, off:off + c] = np.arange(c, dtype=np.float32)
    return jnp.asarray(s), jnp.asarray(t)


@functools.partial(jax.jit, static_argnames=())
def kernel(x, emb0, emb1, emb2, emb3, emb4, emb5, emb6, emb7, emb8, W, b):
    n = x.shape[0]
    s, t = _lane_consts()
    embs = (emb0, emb1, emb2, emb3, emb4, emb5, emb6, emb7, emb8)
    nb = n // BLOCK
    xt = x.T.reshape(9, nb, BLOCK).transpose(1, 0, 2)
    const = lambda i: (0, 0)
    out = pl.pallas_call(
        _main_body,
        grid=(nb,),
        in_specs=[
            pl.BlockSpec((1, 9, BLOCK), lambda i: (i, 0, 0)),
            pl.BlockSpec((9, TOT), const),
            pl.BlockSpec((1, TOT), const),
        ] + [pl.BlockSpec(e.shape, const) for e in embs] + [
            pl.BlockSpec((9 * EMB, HIDDEN), const),
            pl.BlockSpec((1, HIDDEN), const),
        ],
        out_specs=pl.BlockSpec((BLOCK, HIDDEN), lambda i: (i, 0)),
        out_shape=jax.ShapeDtypeStruct((n, HIDDEN), jnp.float32),
        compiler_params=pltpu.CompilerParams(
            dimension_semantics=("arbitrary",)),
    )(xt, s, t, *embs, W, b.reshape(1, HIDDEN))
    return out


# speedup vs baseline: 1.0889x; 1.0585x over previous
import functools

import jax
import jax.numpy as jnp
import numpy as np
from jax.experimental import pallas as pl
from jax.experimental.pallas import tpu as pltpu

CARD = [119, 4, 12, 12, 10, 6, 6, 2, 2]
OFFS = [0, 119, 123, 135, 147, 157, 163, 169, 171]
TOT = 176
EMB = 48
HIDDEN = 256
BLOCK = 25000


def _main_body(x_ref, s_ref, t_ref, e0, e1, e2, e3, e4, e5, e6, e7, e8,
               w_ref, b_ref, o_ref):
    embs = [e0, e1, e2, e3, e4, e5, e6, e7, e8]
    parts = []
    for i in range(9):
        wi = w_ref[EMB * i:EMB * (i + 1), :]
        pi = jnp.dot(embs[i][...], wi, preferred_element_type=jnp.float32,
                     precision=jax.lax.Precision.HIGHEST)
        parts.append(pi)
    parts[0] = parts[0] + b_ref[...]
    parts.append(jnp.zeros((TOT - sum(CARD), HIDDEN), jnp.float32))
    p = jnp.concatenate(parts, axis=0)

    xf = x_ref[0].astype(jnp.float32)
    xg = jax.lax.dot_general(xf, s_ref[...], (((0,), (0,)), ((), ())),
                             preferred_element_type=jnp.float32)
    m = jnp.where(xg == t_ref[...], 1.0, 0.0)
    h = jnp.dot(m, p, preferred_element_type=jnp.float32)
    o_ref[...] = h * 0.5 * (1.0 + jax.lax.erf(h * np.float32(1.0 / np.sqrt(2.0))))


def _lane_consts():
    s = np.zeros((9, TOT), np.float32)
    t = np.full((1, TOT), -1.0, np.float32)
    for i, (off, c) in enumerate(zip(OFFS, CARD)):
        s[i, off:off + c] = 1.0
        t[0, off:off + c] = np.arange(c, dtype=np.float32)
    return jnp.asarray(s), jnp.asarray(t)


@functools.partial(jax.jit, static_argnames=())
def kernel(x, emb0, emb1, emb2, emb3, emb4, emb5, emb6, emb7, emb8, W, b):
    n = x.shape[0]
    s, t = _lane_consts()
    embs = (emb0, emb1, emb2, emb3, emb4, emb5, emb6, emb7, emb8)
    nb = n // BLOCK
    xt = x.astype(jnp.int8).T.reshape(9, nb, BLOCK).transpose(1, 0, 2)
    const = lambda i: (0, 0)
    out = pl.pallas_call(
        _main_body,
        grid=(nb,),
        in_specs=[
            pl.BlockSpec((1, 9, BLOCK), lambda i: (i, 0, 0)),
            pl.BlockSpec((9, TOT), const),
            pl.BlockSpec((1, TOT), const),
        ] + [pl.BlockSpec(e.shape, const) for e in embs] + [
            pl.BlockSpec((9 * EMB, HIDDEN), const),
            pl.BlockSpec((1, HIDDEN), const),
        ],
        out_specs=pl.BlockSpec((BLOCK, HIDDEN), lambda i: (i, 0)),
        out_shape=jax.ShapeDtypeStruct((n, HIDDEN), jnp.float32),
        compiler_params=pltpu.CompilerParams(
            dimension_semantics=("arbitrary",)),
    )(xt, s, t, *embs, W, b.reshape(1, HIDDEN))
    return out
